# baseline (speedup 1.0000x reference)
import functools

import jax
import jax.numpy as jnp
from jax import lax
from jax.experimental import pallas as pl
from jax.experimental.pallas import tpu as pltpu
from jax.experimental.pallas import tpu_sc as plsc

NC = 2
NS = 16
LANES = 16


def _make_seg_sum(n, dh, nblk, kb, c, with_counts, nbuf):
  rpt = 8 * (n // (8 * NS))
  tail = n - rpt * NS

  mesh = plsc.VectorSubcoreMesh(core_axis_name="c", subcore_axis_name="s",
                                num_cores=NC)

  p_type = jax.ShapeDtypeStruct((NC, n, dh), jnp.float32)
  if with_counts:
    out_type = [p_type, jax.ShapeDtypeStruct((NC, n, LANES), jnp.float32)]
  else:
    out_type = p_type

  scratch = (
      [pltpu.VMEM((kb, c), jnp.int32)] * 2
      + [pltpu.VMEM((c, dh), jnp.float32)] * nbuf
      + ([pltpu.VMEM((c, LANES), jnp.float32),
          pltpu.VMEM_SHARED((n, dh), jnp.float32),
          pltpu.VMEM_SHARED((n, LANES), jnp.float32)]
         if with_counts else
         [pltpu.VMEM_SHARED((n, dh), jnp.float32)])
      + [pltpu.SemaphoreType.DMA] * (2 * nbuf + 2 + (1 if with_counts else 0))
  )

  @functools.partial(
      pl.kernel, out_type=out_type, mesh=mesh, scratch_types=scratch,
      compiler_params=pltpu.CompilerParams(use_tc_tiling_on_sc=False))
  def seg_sum(h_hbm, src_hbm, dst_hbm, zrows_hbm, zcnt_hbm, ones_hbm,
              *out_and_scratch):
    if with_counts:
      p_hbm, cnt_hbm = out_and_scratch[0], out_and_scratch[1]
      (src_v, dst_v, *rest) = out_and_scratch[2:]
      rows = rest[:nbuf]
      ones_v, acc_sh, cnt_sh = rest[nbuf:nbuf + 3]
      sems = rest[nbuf + 3:]
    else:
      p_hbm = out_and_scratch[0]
      cnt_hbm = cnt_sh = ones_v = None
      (src_v, dst_v, *rest) = out_and_scratch[1:]
      rows = rest[:nbuf]
      acc_sh = rest[nbuf]
      sems = rest[nbuf + 1:]
    gsem = sems[:nbuf]
    ssem = sems[nbuf:2 * nbuf]
    isem_s, isem_d = sems[2 * nbuf:2 * nbuf + 2]
    csem = sems[2 * nbuf + 2] if with_counts else None

    cid = lax.axis_index("c")
    sid = lax.axis_index("s")
    htab = h_hbm.at[cid]

    pltpu.sync_copy(zrows_hbm, acc_sh.at[pl.ds(sid * rpt, rpt)])
    if with_counts:
      pltpu.sync_copy(zcnt_hbm, cnt_sh.at[pl.ds(sid * rpt, rpt)])
      pltpu.sync_copy(ones_hbm, ones_v)
    if tail:
      @pl.when(sid == 0)
      def _():
        pltpu.sync_copy(zrows_hbm.at[pl.ds(0, tail)],
                        acc_sh.at[pl.ds(NS * rpt, tail)])
        if with_counts:
          pltpu.sync_copy(zcnt_hbm.at[pl.ds(0, tail)],
                          cnt_sh.at[pl.ds(NS * rpt, tail)])
    plsc.subcore_barrier()

    def block(j, carry):
      if with_counts:
        do_cnt = jnp.logical_or(
            jnp.logical_and(cid == 0, j < nblk // 2),
            jnp.logical_and(cid == 1, j >= nblk // 2))

      pltpu.async_copy(src_hbm.at[sid, j], src_v, isem_s)
      pltpu.async_copy(dst_hbm.at[sid, j], dst_v, isem_d)
      pltpu.make_async_copy(src_hbm.at[sid, j], src_v, isem_s).wait()

      for b in range(nbuf):
        pltpu.async_copy(htab.at[src_v.at[b]], rows[b], gsem[b])
      pltpu.make_async_copy(dst_hbm.at[sid, j], dst_v, isem_d).wait()

      def chunk(k, b, prefetch):
        pltpu.make_async_copy(htab.at[src_v.at[k]], rows[b],
                              gsem[b]).wait()
        pltpu.async_copy(rows[b], acc_sh.at[dst_v.at[k]], ssem[b],
                         add=True)
        if with_counts:
          @pl.when(do_cnt)
          def _():
            pltpu.async_copy(ones_v, cnt_sh.at[dst_v.at[k]], csem,
                             add=True)
        pltpu.make_async_copy(rows[b], acc_sh.at[dst_v.at[k]],
                              ssem[b]).wait()
        if prefetch:
          pltpu.async_copy(htab.at[src_v.at[k + nbuf]], rows[b], gsem[b])

      def step(i2, carry2):
        for b in range(nbuf):
          chunk(i2 * nbuf + b, b, prefetch=True)
        return carry2

      lax.fori_loop(0, kb // nbuf - 1, step, 0)
      for b in range(nbuf):
        chunk(kb - nbuf + b, b, prefetch=False)

      if with_counts:
        @pl.when(do_cnt)
        def _():
          def drain(k, carry3):
            pltpu.make_async_copy(ones_v, cnt_sh.at[dst_v.at[0]],
                                  csem).wait()
            return carry3
          lax.fori_loop(0, kb, drain, 0)
      return carry

    lax.fori_loop(0, nblk, block, 0)

    plsc.subcore_barrier()
    pltpu.sync_copy(acc_sh.at[pl.ds(sid * rpt, rpt)],
                    p_hbm.at[cid, pl.ds(sid * rpt, rpt)])
    if with_counts:
      pltpu.sync_copy(cnt_sh.at[pl.ds(sid * rpt, rpt)],
                      cnt_hbm.at[cid, pl.ds(sid * rpt, rpt)])
    if tail:
      @pl.when(sid == 0)
      def _():
        pltpu.sync_copy(acc_sh.at[pl.ds(NS * rpt, tail)],
                        p_hbm.at[cid, pl.ds(NS * rpt, tail)])
        if with_counts:
          pltpu.sync_copy(cnt_sh.at[pl.ds(NS * rpt, tail)],
                          cnt_hbm.at[cid, pl.ds(NS * rpt, tail)])

    return None

  return seg_sum


def _relu_split_body(x_ref, o_ref, *, dh):
  h = jnp.maximum(x_ref[...], 0.0)
  o_ref[0] = h[:, :dh]
  o_ref[1] = h[:, dh:]


def _relu_split_tc(x, dh):
  n, d = x.shape
  bn = 2000
  return pl.pallas_call(
      functools.partial(_relu_split_body, dh=dh),
      grid=(n // bn,),
      in_specs=[pl.BlockSpec((bn, d), lambda i: (i, 0))],
      out_specs=pl.BlockSpec((NC, bn, dh), lambda i: (0, i, 0)),
      out_shape=jax.ShapeDtypeStruct((NC, n, dh), jnp.float32),
  )(x)


def _combine_body(p_ref, cnt_ref, h_ref, wna_ref, wnb_ref, bn_ref,
                  wsa_ref, wsb_ref, bs_ref, wuna_ref, wunb_ref,
                  wusa_ref, wusb_ref, bua_ref, bub_ref, o_ref, *, relu):
  dot = functools.partial(jnp.dot, preferred_element_type=jnp.float32)
  cden = jnp.maximum(cnt_ref[0, :, 0] + cnt_ref[1, :, 0], 1.0)[:, None]
  m0 = p_ref[0] / cden
  m1 = p_ref[1] / cden
  hn = dot(m0, wna_ref[...]) + dot(m1, wnb_ref[...]) + bn_ref[0]
  hs = dot(h_ref[0], wsa_ref[...]) + dot(h_ref[1], wsb_ref[...]) + bs_ref[0]
  ua = dot(hn, wuna_ref[...]) + dot(hs, wusa_ref[...]) + bua_ref[0]
  ub = dot(hn, wunb_ref[...]) + dot(hs, wusb_ref[...]) + bub_ref[0]
  if relu:
    o_ref[0] = jnp.maximum(ua, 0.0)
    o_ref[1] = jnp.maximum(ub, 0.0)
  else:
    o_ref[...] = jnp.concatenate([ua, ub], axis=1)


def _combine_tc(p, cnt, h2, wn, bn, ws, bs, wu, bu, relu):
  _, n, dh = h2.shape
  d = 2 * dh
  hdim = wn.shape[1]
  bn_ = bn.reshape(1, hdim)
  bs_ = bs.reshape(1, hdim)
  wna, wnb = wn[:dh], wn[dh:]
  wsa, wsb = ws[:dh], ws[dh:]
  wun, wus = wu[:hdim], wu[hdim:]
  wuna, wunb = wun[:, :dh], wun[:, dh:]
  wusa, wusb = wus[:, :dh], wus[:, dh:]
  bua = bu[:dh].reshape(1, dh)
  bub = bu[dh:].reshape(1, dh)
  bnrows = 2000
  grid = (n // bnrows,)
  if relu:
    out_shape = jax.ShapeDtypeStruct((NC, n, dh), jnp.float32)
    out_spec = pl.BlockSpec((NC, bnrows, dh), lambda i: (0, i, 0))
  else:
    out_shape = jax.ShapeDtypeStruct((n, hdim), jnp.float32)
    out_spec = pl.BlockSpec((bnrows, hdim), lambda i: (i, 0))
  full = lambda a: pl.BlockSpec(a.shape, lambda i: (0,) * a.ndim)
  return pl.pallas_call(
      functools.partial(_combine_body, relu=relu),
      grid=grid,
      in_specs=[
          pl.BlockSpec((NC, bnrows, dh), lambda i: (0, i, 0)),
          pl.BlockSpec((NC, bnrows, LANES), lambda i: (0, i, 0)),
          pl.BlockSpec((NC, bnrows, dh), lambda i: (0, i, 0)),
          full(wna), full(wnb), full(bn_),
          full(wsa), full(wsb), full(bs_),
          full(wuna), full(wunb), full(wusa), full(wusb),
          full(bua), full(bub),
      ],
      out_specs=out_spec,
      out_shape=out_shape,
  )(p, cnt, h2, wna, wnb, bn_, wsa, wsb, bs_, wuna, wunb, wusa, wusb,
    bua, bub)


def kernel(x, edge_index, Wn1, bn1, Ws1, bs1, Wu1, bu1,
           Wn2, bn2, Ws2, bs2, Wu2, bu2):
  n, d = x.shape
  e = edge_index.shape[1]
  dh = d // NC
  ept = e // NS
  c1, kb1, r1 = 40, 100, 4
  c2, kb2, r2 = 40, 100, 5
  nblk1 = ept // (kb1 * c1)
  nblk2 = ept // (kb2 * c2)
  assert dh * NC == d and ept * NS == e
  assert nblk1 * kb1 * c1 == ept and nblk2 * kb2 * c2 == ept
  assert n % NS == 0

  src1 = edge_index[0].reshape(NS, nblk1, kb1, c1)
  dst1 = edge_index[1].reshape(NS, nblk1, kb1, c1)
  src2 = edge_index[0].reshape(NS, nblk2, kb2, c2)
  dst2 = edge_index[1].reshape(NS, nblk2, kb2, c2)
  rpt = 8 * (n // (8 * NS))
  assert 0 <= n - rpt * NS <= rpt
  zrows = jnp.zeros((rpt, dh), jnp.float32)
  zcnt = jnp.zeros((rpt, LANES), jnp.float32)
  ones = jnp.ones((c1, LANES), jnp.float32)

  seg1 = _make_seg_sum(n, dh, nblk1, kb1, c1, with_counts=True, nbuf=r1)
  seg2 = _make_seg_sum(n, dh, nblk2, kb2, c2, with_counts=False, nbuf=r2)

  h1 = _relu_split_tc(x, dh)
  p1, cnt = seg1(h1, src1, dst1, zrows, zcnt, ones)
  h2 = _combine_tc(p1, cnt, h1, Wn1, bn1, Ws1, bs1, Wu1, bu1, relu=True)
  p2 = seg2(h2, src2, dst2, zrows, zcnt, ones)
  out = _combine_tc(p2, cnt, h2, Wn2, bn2, Ws2, bs2, Wu2, bu2, relu=False)
  return out


# speedup vs baseline: 1.0291x; 1.0291x over previous
import functools

import jax
import jax.numpy as jnp
from jax import lax
from jax.experimental import pallas as pl
from jax.experimental.pallas import tpu as pltpu
from jax.experimental.pallas import tpu_sc as plsc

NC = 2
NS = 16
LANES = 16


def _make_seg_sum(n, dh, nblk, kb, c, with_counts, nbuf):
  rpt = 8 * (n // (8 * NS))
  tail = n - rpt * NS

  mesh = plsc.VectorSubcoreMesh(core_axis_name="c", subcore_axis_name="s",
                                num_cores=NC)

  p_type = jax.ShapeDtypeStruct((NC, n, dh), jnp.float32)
  if with_counts:
    out_type = [p_type, jax.ShapeDtypeStruct((NC, n, LANES), jnp.float32)]
  else:
    out_type = p_type

  scratch = (
      [pltpu.VMEM((kb, c), jnp.int32)] * 2
      + [pltpu.VMEM((c, dh), jnp.float32)] * nbuf
      + ([pltpu.VMEM((c, LANES), jnp.float32),
          pltpu.VMEM_SHARED((n, dh), jnp.float32),
          pltpu.VMEM_SHARED((n, LANES), jnp.float32)]
         if with_counts else
         [pltpu.VMEM_SHARED((n, dh), jnp.float32)])
      + [pltpu.SemaphoreType.DMA] * (2 * nbuf + 2 + (1 if with_counts else 0))
  )

  @functools.partial(
      pl.kernel, out_type=out_type, mesh=mesh, scratch_types=scratch,
      compiler_params=pltpu.CompilerParams(use_tc_tiling_on_sc=False))
  def seg_sum(h_hbm, src_hbm, dst_hbm, zrows_hbm, zcnt_hbm, ones_hbm,
              *out_and_scratch):
    if with_counts:
      p_hbm, cnt_hbm = out_and_scratch[0], out_and_scratch[1]
      (src_v, dst_v, *rest) = out_and_scratch[2:]
      rows = rest[:nbuf]
      ones_v, acc_sh, cnt_sh = rest[nbuf:nbuf + 3]
      sems = rest[nbuf + 3:]
    else:
      p_hbm = out_and_scratch[0]
      cnt_hbm = cnt_sh = ones_v = None
      (src_v, dst_v, *rest) = out_and_scratch[1:]
      rows = rest[:nbuf]
      acc_sh = rest[nbuf]
      sems = rest[nbuf + 1:]
    gsem = sems[:nbuf]
    ssem = sems[nbuf:2 * nbuf]
    isem_s, isem_d = sems[2 * nbuf:2 * nbuf + 2]
    csem = sems[2 * nbuf + 2] if with_counts else None

    cid = lax.axis_index("c")
    sid = lax.axis_index("s")
    htab = h_hbm.at[cid]

    pltpu.sync_copy(zrows_hbm, acc_sh.at[pl.ds(sid * rpt, rpt)])
    if with_counts:
      pltpu.sync_copy(zcnt_hbm, cnt_sh.at[pl.ds(sid * rpt, rpt)])
      pltpu.sync_copy(ones_hbm, ones_v)
    if tail:
      @pl.when(sid == 0)
      def _():
        pltpu.sync_copy(zrows_hbm.at[pl.ds(0, tail)],
                        acc_sh.at[pl.ds(NS * rpt, tail)])
        if with_counts:
          pltpu.sync_copy(zcnt_hbm.at[pl.ds(0, tail)],
                          cnt_sh.at[pl.ds(NS * rpt, tail)])
    plsc.subcore_barrier()

    def block(j, carry):
      if with_counts:
        do_cnt = jnp.logical_or(
            jnp.logical_and(cid == 0, j < nblk // 2),
            jnp.logical_and(cid == 1, j >= nblk // 2))

      pltpu.async_copy(src_hbm.at[sid, j], src_v, isem_s)
      pltpu.async_copy(dst_hbm.at[sid, j], dst_v, isem_d)
      pltpu.make_async_copy(src_hbm.at[sid, j], src_v, isem_s).wait()

      for b in range(nbuf):
        pltpu.async_copy(htab.at[src_v.at[b]], rows[b], gsem[b])
      pltpu.make_async_copy(dst_hbm.at[sid, j], dst_v, isem_d).wait()

      def chunk(k, b, prefetch):
        pltpu.make_async_copy(htab.at[src_v.at[k]], rows[b],
                              gsem[b]).wait()
        pltpu.async_copy(rows[b], acc_sh.at[dst_v.at[k]], ssem[b],
                         add=True)
        if with_counts:
          @pl.when(do_cnt)
          def _():
            pltpu.async_copy(ones_v, cnt_sh.at[dst_v.at[k]], csem,
                             add=True)
        pltpu.make_async_copy(rows[b], acc_sh.at[dst_v.at[k]],
                              ssem[b]).wait()
        if prefetch:
          pltpu.async_copy(htab.at[src_v.at[k + nbuf]], rows[b], gsem[b])

      def step(i2, carry2):
        for b in range(nbuf):
          chunk(i2 * nbuf + b, b, prefetch=True)
        return carry2

      lax.fori_loop(0, kb // nbuf - 1, step, 0)
      for b in range(nbuf):
        chunk(kb - nbuf + b, b, prefetch=False)

      if with_counts:
        @pl.when(do_cnt)
        def _():
          def drain(k, carry3):
            pltpu.make_async_copy(ones_v, cnt_sh.at[dst_v.at[0]],
                                  csem).wait()
            return carry3
          lax.fori_loop(0, kb, drain, 0)
      return carry

    lax.fori_loop(0, nblk, block, 0)

    plsc.subcore_barrier()
    pltpu.sync_copy(acc_sh.at[pl.ds(sid * rpt, rpt)],
                    p_hbm.at[cid, pl.ds(sid * rpt, rpt)])
    if with_counts:
      pltpu.sync_copy(cnt_sh.at[pl.ds(sid * rpt, rpt)],
                      cnt_hbm.at[cid, pl.ds(sid * rpt, rpt)])
    if tail:
      @pl.when(sid == 0)
      def _():
        pltpu.sync_copy(acc_sh.at[pl.ds(NS * rpt, tail)],
                        p_hbm.at[cid, pl.ds(NS * rpt, tail)])
        if with_counts:
          pltpu.sync_copy(cnt_sh.at[pl.ds(NS * rpt, tail)],
                          cnt_hbm.at[cid, pl.ds(NS * rpt, tail)])

    return None

  return seg_sum


def _relu_split_body(x_ref, o_ref, *, dh):
  h = jnp.maximum(x_ref[...], 0.0)
  o_ref[0] = h[:, :dh]
  o_ref[1] = h[:, dh:]


def _relu_split_tc(x, dh):
  n, d = x.shape
  bn = 2000
  return pl.pallas_call(
      functools.partial(_relu_split_body, dh=dh),
      grid=(n // bn,),
      in_specs=[pl.BlockSpec((bn, d), lambda i: (i, 0))],
      out_specs=pl.BlockSpec((NC, bn, dh), lambda i: (0, i, 0)),
      out_shape=jax.ShapeDtypeStruct((NC, n, dh), jnp.float32),
  )(x)


def _combine_body(p_ref, cnt_ref, h_ref, wna_ref, wnb_ref, bn_ref,
                  wsa_ref, wsb_ref, bs_ref, wuna_ref, wunb_ref,
                  wusa_ref, wusb_ref, bua_ref, bub_ref, o_ref, *, relu):
  dot = functools.partial(jnp.dot, preferred_element_type=jnp.float32)
  cden = jnp.maximum(cnt_ref[0, :, 0] + cnt_ref[1, :, 0], 1.0)[:, None]
  m0 = p_ref[0] / cden
  m1 = p_ref[1] / cden
  hn = dot(m0, wna_ref[...]) + dot(m1, wnb_ref[...]) + bn_ref[0]
  hs = dot(h_ref[0], wsa_ref[...]) + dot(h_ref[1], wsb_ref[...]) + bs_ref[0]
  ua = dot(hn, wuna_ref[...]) + dot(hs, wusa_ref[...]) + bua_ref[0]
  ub = dot(hn, wunb_ref[...]) + dot(hs, wusb_ref[...]) + bub_ref[0]
  if relu:
    o_ref[0] = jnp.maximum(ua, 0.0)
    o_ref[1] = jnp.maximum(ub, 0.0)
  else:
    o_ref[...] = jnp.concatenate([ua, ub], axis=1)


def _combine_tc(p, cnt, h2, wn, bn, ws, bs, wu, bu, relu):
  _, n, dh = h2.shape
  d = 2 * dh
  hdim = wn.shape[1]
  bn_ = bn.reshape(1, hdim)
  bs_ = bs.reshape(1, hdim)
  wna, wnb = wn[:dh], wn[dh:]
  wsa, wsb = ws[:dh], ws[dh:]
  wun, wus = wu[:hdim], wu[hdim:]
  wuna, wunb = wun[:, :dh], wun[:, dh:]
  wusa, wusb = wus[:, :dh], wus[:, dh:]
  bua = bu[:dh].reshape(1, dh)
  bub = bu[dh:].reshape(1, dh)
  bnrows = 2000
  grid = (n // bnrows,)
  if relu:
    out_shape = jax.ShapeDtypeStruct((NC, n, dh), jnp.float32)
    out_spec = pl.BlockSpec((NC, bnrows, dh), lambda i: (0, i, 0))
  else:
    out_shape = jax.ShapeDtypeStruct((n, hdim), jnp.float32)
    out_spec = pl.BlockSpec((bnrows, hdim), lambda i: (i, 0))
  full = lambda a: pl.BlockSpec(a.shape, lambda i: (0,) * a.ndim)
  return pl.pallas_call(
      functools.partial(_combine_body, relu=relu),
      grid=grid,
      in_specs=[
          pl.BlockSpec((NC, bnrows, dh), lambda i: (0, i, 0)),
          pl.BlockSpec((NC, bnrows, LANES), lambda i: (0, i, 0)),
          pl.BlockSpec((NC, bnrows, dh), lambda i: (0, i, 0)),
          full(wna), full(wnb), full(bn_),
          full(wsa), full(wsb), full(bs_),
          full(wuna), full(wunb), full(wusa), full(wusb),
          full(bua), full(bub),
      ],
      out_specs=out_spec,
      out_shape=out_shape,
  )(p, cnt, h2, wna, wnb, bn_, wsa, wsb, bs_, wuna, wunb, wusa, wusb,
    bua, bub)


def kernel(x, edge_index, Wn1, bn1, Ws1, bs1, Wu1, bu1,
           Wn2, bn2, Ws2, bs2, Wu2, bu2):
  n, d = x.shape
  e = edge_index.shape[1]
  dh = d // NC
  ept = e // NS
  c1, kb1, r1 = 40, 50, 5
  c2, kb2, r2 = 40, 100, 5
  nblk1 = ept // (kb1 * c1)
  nblk2 = ept // (kb2 * c2)
  assert dh * NC == d and ept * NS == e
  assert nblk1 * kb1 * c1 == ept and nblk2 * kb2 * c2 == ept
  assert n % NS == 0

  src1 = edge_index[0].reshape(NS, nblk1, kb1, c1)
  dst1 = edge_index[1].reshape(NS, nblk1, kb1, c1)
  src2 = edge_index[0].reshape(NS, nblk2, kb2, c2)
  dst2 = edge_index[1].reshape(NS, nblk2, kb2, c2)
  rpt = 8 * (n // (8 * NS))
  assert 0 <= n - rpt * NS <= rpt
  zrows = jnp.zeros((rpt, dh), jnp.float32)
  zcnt = jnp.zeros((rpt, LANES), jnp.float32)
  ones = jnp.ones((c1, LANES), jnp.float32)

  seg1 = _make_seg_sum(n, dh, nblk1, kb1, c1, with_counts=True, nbuf=r1)
  seg2 = _make_seg_sum(n, dh, nblk2, kb2, c2, with_counts=False, nbuf=r2)

  h1 = _relu_split_tc(x, dh)
  p1, cnt = seg1(h1, src1, dst1, zrows, zcnt, ones)
  h2 = _combine_tc(p1, cnt, h1, Wn1, bn1, Ws1, bs1, Wu1, bu1, relu=True)
  p2 = seg2(h2, src2, dst2, zrows, zcnt, ones)
  out = _combine_tc(p2, cnt, h2, Wn2, bn2, Ws2, bs2, Wu2, bu2, relu=False)
  return out
